# initial kernel scaffold (unmeasured)
import jax
import jax.numpy as jnp
from jax import lax
from jax.experimental import pallas as pl
from jax.experimental.pallas import tpu as pltpu

N_DEV = 4
S = 2048
D = 1024
HN = 8
DH = 128
BLK = 64
NB = S // BLK
R = 4
J = NB // R
G = J * BLK
CH = S // N_DEV
SCALE = 0.08838834764831843


def _body(x_ref, wq_ref, k_ref, v_ref, wo_ref, out_ref,
          qp, kp, vp, cp, comm, send_sems, recv_sems):
    my = lax.axis_index("i")
    right = jnp.mod(my + 1, N_DEV)
    left = jnp.mod(my + N_DEV - 1, N_DEV)

    for b in range(NB):
        r, j = b % R, b // R
        d0 = r * G + j * BLK
        s0 = b * BLK
        kp[d0:d0 + BLK, :] = k_ref[s0:s0 + BLK, :]
        vp[d0:d0 + BLK, :] = v_ref[s0:s0 + BLK, :]

    for c in range(N_DEV):
        qf = lax.dot_general(
            x_ref[c * CH:(c + 1) * CH, :], wq_ref[:, :],
            (((1,), (0,)), ((), ())), preferred_element_type=jnp.float32)
        qb = (qf * SCALE).astype(jnp.bfloat16)
        for m in range(CH // BLK):
            b = (CH // BLK) * c + m
            r, j = b % R, b // R
            d0 = r * G + j * BLK
            qp[d0:d0 + BLK, :] = qb[m * BLK:(m + 1) * BLK, :]

    for r in range(R):
        for h in range(HN):
            rows = slice(r * G, (r + 1) * G)
            cols = slice(h * DH, (h + 1) * DH)
            q = qp[rows, cols]
            k = kp[rows, cols]
            v = vp[rows, cols]
            s = lax.dot_general(q, k, (((1,), (1,)), ((), ())),
                                preferred_element_type=jnp.float32)
            mx = jnp.max(s, axis=1, keepdims=True)
            w = jnp.exp(s - mx)
            w = w / jnp.sum(w, axis=1, keepdims=True)
            ctx = lax.dot_general(w.astype(jnp.bfloat16), v,
                                  (((1,), (0,)), ((), ())),
                                  preferred_element_type=jnp.float32)
            cp[rows, cols] = ctx.astype(jnp.bfloat16)

    for r in range(R):
        p = lax.dot_general(cp[r * G:(r + 1) * G, :], wo_ref[:, :],
                            (((1,), (0,)), ((), ())),
                            preferred_element_type=jnp.float32)
        for j in range(J):
            b = R * j + r
            out_ref[0, b * BLK:(b + 1) * BLK, :] = p[j * BLK:(j + 1) * BLK, :]

    bsem = pltpu.get_barrier_semaphore()
    for nbr in (left, right):
        pl.semaphore_signal(bsem, inc=1, device_id=(nbr,),
                            device_id_type=pl.DeviceIdType.MESH)
    pl.semaphore_wait(bsem, 2)

    for st in range(N_DEV - 1):
        send_chunk = jnp.mod(my - st, N_DEV)
        if st == 0:
            src = out_ref.at[0, pl.ds(send_chunk * CH, CH), :]
        else:
            src = comm.at[st - 1]
        rdma = pltpu.make_async_remote_copy(
            src_ref=src,
            dst_ref=comm.at[st],
            send_sem=send_sems.at[st],
            recv_sem=recv_sems.at[st],
            device_id=(right,),
            device_id_type=pl.DeviceIdType.MESH,
        )
        rdma.start()
        rdma.wait()
        rc = jnp.mod(my - st - 1, N_DEV)
        acc = comm[st] + out_ref[0, pl.ds(rc * CH, CH), :]
        if st < N_DEV - 2:
            comm[st] = acc
        else:
            out_ref[0, pl.ds(rc * CH, CH), :] = acc

    for t in range(N_DEV - 1):
        ct = jnp.mod(my + 1 - t, N_DEV)
        rdma = pltpu.make_async_remote_copy(
            src_ref=out_ref.at[0, pl.ds(ct * CH, CH), :],
            dst_ref=out_ref.at[0, pl.ds(ct * CH, CH), :],
            send_sem=send_sems.at[N_DEV - 1 + t],
            recv_sem=recv_sems.at[N_DEV - 1 + t],
            device_id=(right,),
            device_id_type=pl.DeviceIdType.MESH,
        )
        rdma.start()
        rdma.wait()


def kernel(x, Wq, K_ext, V_ext, Wo):
    my = lax.axis_index("i")
    xb = x[0].astype(jnp.bfloat16)
    wq = Wq.astype(jnp.bfloat16)
    wo = Wo.astype(jnp.bfloat16)
    kh = lax.dynamic_slice_in_dim(K_ext[0], my * HN, HN, axis=1)
    vh = lax.dynamic_slice_in_dim(V_ext[0], my * HN, HN, axis=1)
    kb = kh.reshape(S, HN * DH).astype(jnp.bfloat16)
    vb = vh.reshape(S, HN * DH).astype(jnp.bfloat16)

    return pl.pallas_call(
        _body,
        out_shape=jax.ShapeDtypeStruct((1, S, D), jnp.float32),
        in_specs=[pl.BlockSpec(memory_space=pltpu.VMEM)] * 5,
        out_specs=pl.BlockSpec(memory_space=pltpu.VMEM),
        scratch_shapes=[
            pltpu.VMEM((S, D), jnp.bfloat16),
            pltpu.VMEM((S, D), jnp.bfloat16),
            pltpu.VMEM((S, D), jnp.bfloat16),
            pltpu.VMEM((S, D), jnp.bfloat16),
            pltpu.VMEM((2 * (N_DEV - 1), CH, D), jnp.float32),
            pltpu.SemaphoreType.DMA((2 * (N_DEV - 1),)),
            pltpu.SemaphoreType.DMA((2 * (N_DEV - 1),)),
        ],
        compiler_params=pltpu.CompilerParams(collective_id=0),
    )(xb, wq, kb, vb, wo)


# baseline (device time: 215164 ns/iter reference)
import jax
import jax.numpy as jnp
from jax import lax
from jax.experimental import pallas as pl
from jax.experimental.pallas import tpu as pltpu

N_DEV = 4
S = 2048
D = 1024
HN = 8
DH = 128
BLK = 64
NB = S // BLK
R = 4
J = NB // R
G = J * BLK
CH = S // N_DEV
SCALE = 0.08838834764831843


def _body(x_ref, wq_ref, k_ref, v_ref, wo_ref, out_ref,
          qp, kp, vp, cp, comm, send_sems, recv_sems):
    my = lax.axis_index("i")
    right = jnp.mod(my + 1, N_DEV)
    left = jnp.mod(my + N_DEV - 1, N_DEV)

    for b in range(NB):
        r, j = b % R, b // R
        d0 = r * G + j * BLK
        s0 = b * BLK
        kp[d0:d0 + BLK, :] = k_ref[s0:s0 + BLK, :]
        vp[d0:d0 + BLK, :] = v_ref[s0:s0 + BLK, :]

    for c in range(N_DEV):
        qf = lax.dot_general(
            x_ref[c * CH:(c + 1) * CH, :], wq_ref[:, :],
            (((1,), (0,)), ((), ())), preferred_element_type=jnp.float32)
        qb = (qf * SCALE).astype(jnp.bfloat16)
        for m in range(CH // BLK):
            b = (CH // BLK) * c + m
            r, j = b % R, b // R
            d0 = r * G + j * BLK
            qp[d0:d0 + BLK, :] = qb[m * BLK:(m + 1) * BLK, :]

    for r in range(R):
        for h in range(HN):
            rows = slice(r * G, (r + 1) * G)
            cols = slice(h * DH, (h + 1) * DH)
            q = qp[rows, cols]
            k = kp[rows, cols]
            v = vp[rows, cols]
            s = lax.dot_general(q, k, (((1,), (1,)), ((), ())),
                                preferred_element_type=jnp.float32)
            mx = jnp.max(s, axis=1, keepdims=True)
            w = jnp.exp(s - mx)
            w = w / jnp.sum(w, axis=1, keepdims=True)
            ctx = lax.dot_general(w.astype(jnp.bfloat16), v,
                                  (((1,), (0,)), ((), ())),
                                  preferred_element_type=jnp.float32)
            cp[rows, cols] = ctx.astype(jnp.bfloat16)

    for r in range(R):
        p = lax.dot_general(cp[r * G:(r + 1) * G, :], wo_ref[:, :],
                            (((1,), (0,)), ((), ())),
                            preferred_element_type=jnp.float32)
        for j in range(J):
            b = R * j + r
            out_ref[0, b * BLK:(b + 1) * BLK, :] = p[j * BLK:(j + 1) * BLK, :]

    bsem = pltpu.get_barrier_semaphore()
    for nbr in (left, right):
        pl.semaphore_signal(bsem, inc=1, device_id=(nbr,),
                            device_id_type=pl.DeviceIdType.MESH)
    pl.semaphore_wait(bsem, 2)

    for st in range(N_DEV - 1):
        send_chunk = jnp.mod(my - st, N_DEV)
        if st == 0:
            src = out_ref.at[0, pl.ds(send_chunk * CH, CH), :]
        else:
            src = comm.at[st - 1]
        rdma = pltpu.make_async_remote_copy(
            src_ref=src,
            dst_ref=comm.at[st],
            send_sem=send_sems.at[st],
            recv_sem=recv_sems.at[st],
            device_id=(right,),
            device_id_type=pl.DeviceIdType.MESH,
        )
        rdma.start()
        rdma.wait()
        rc = jnp.mod(my - st - 1, N_DEV)
        acc = comm[st] + out_ref[0, pl.ds(rc * CH, CH), :]
        if st < N_DEV - 2:
            comm[st] = acc
        else:
            out_ref[0, pl.ds(rc * CH, CH), :] = acc

    for t in range(N_DEV - 1):
        ct = jnp.mod(my + 1 - t, N_DEV)
        rdma = pltpu.make_async_remote_copy(
            src_ref=out_ref.at[0, pl.ds(ct * CH, CH), :],
            dst_ref=out_ref.at[0, pl.ds(ct * CH, CH), :],
            send_sem=send_sems.at[N_DEV - 1 + t],
            recv_sem=recv_sems.at[N_DEV - 1 + t],
            device_id=(right,),
            device_id_type=pl.DeviceIdType.MESH,
        )
        rdma.start()
        rdma.wait()


def kernel(x, Wq, K_ext, V_ext, Wo):
    my = lax.axis_index("i")
    xb = x[0].astype(jnp.bfloat16)
    wq = Wq.astype(jnp.bfloat16)
    wo = Wo.astype(jnp.bfloat16)
    kh = lax.dynamic_slice_in_dim(K_ext[0], my * HN, HN, axis=1)
    vh = lax.dynamic_slice_in_dim(V_ext[0], my * HN, HN, axis=1)
    kb = kh.reshape(S, HN * DH).astype(jnp.bfloat16)
    vb = vh.reshape(S, HN * DH).astype(jnp.bfloat16)

    return pl.pallas_call(
        _body,
        out_shape=jax.ShapeDtypeStruct((1, S, D), jnp.float32),
        in_specs=[pl.BlockSpec(memory_space=pltpu.VMEM)] * 5,
        out_specs=pl.BlockSpec(memory_space=pltpu.VMEM),
        scratch_shapes=[
            pltpu.VMEM((S, D), jnp.bfloat16),
            pltpu.VMEM((S, D), jnp.bfloat16),
            pltpu.VMEM((S, D), jnp.bfloat16),
            pltpu.VMEM((S, D), jnp.bfloat16),
            pltpu.VMEM((2 * (N_DEV - 1), CH, D), jnp.float32),
            pltpu.SemaphoreType.DMA((2 * (N_DEV - 1),)),
            pltpu.SemaphoreType.DMA((2 * (N_DEV - 1),)),
        ],
        compiler_params=pltpu.CompilerParams(
            collective_id=0, vmem_limit_bytes=63 * 1024 * 1024),
    )(xb, wq, kb, vb, wo)


# device time: 147784 ns/iter; 1.4559x vs baseline; 1.4559x over previous
import jax
import jax.numpy as jnp
from jax import lax
from jax.experimental import pallas as pl
from jax.experimental.pallas import tpu as pltpu

N_DEV = 4
S = 2048
D = 1024
HN = 8
DH = 128
BLK = 64
NB = S // BLK
R = 4
J = NB // R
G = J * BLK
CH = S // N_DEV
SCALE = 0.08838834764831843


def _body(x_ref, wq_ref, k_ref, v_ref, wo_ref, out_ref,
          qp, kp, vp, cp, sb, rs, ag, send_sems, recv_sems):
    my = lax.axis_index("i")
    right = jnp.mod(my + 1, N_DEV)
    left = jnp.mod(my + N_DEV - 1, N_DEV)

    for b in range(NB):
        r, j = b % R, b // R
        d0 = r * G + j * BLK
        s0 = b * BLK
        kp[d0:d0 + BLK, :] = k_ref[s0:s0 + BLK, :]
        vp[d0:d0 + BLK, :] = v_ref[s0:s0 + BLK, :]

    for c in range(N_DEV):
        qf = lax.dot_general(
            x_ref[c * CH:(c + 1) * CH, :], wq_ref[:, :],
            (((1,), (0,)), ((), ())), preferred_element_type=jnp.float32)
        qb = (qf * SCALE).astype(jnp.bfloat16)
        for m in range(CH // BLK):
            b = (CH // BLK) * c + m
            r, j = b % R, b // R
            d0 = r * G + j * BLK
            qp[d0:d0 + BLK, :] = qb[m * BLK:(m + 1) * BLK, :]

    for r in range(R):
        for h in range(HN):
            rows = slice(r * G, (r + 1) * G)
            cols = slice(h * DH, (h + 1) * DH)
            q = qp[rows, cols]
            k = kp[rows, cols]
            v = vp[rows, cols]
            s = lax.dot_general(q, k, (((1,), (1,)), ((), ())),
                                preferred_element_type=jnp.float32)
            mx = jnp.max(s, axis=1, keepdims=True)
            w = jnp.exp(s - mx)
            w = w / jnp.sum(w, axis=1, keepdims=True)
            ctx = lax.dot_general(w.astype(jnp.bfloat16), v,
                                  (((1,), (0,)), ((), ())),
                                  preferred_element_type=jnp.float32)
            cp[rows, cols] = ctx.astype(jnp.bfloat16)

    for r in range(R):
        p = lax.dot_general(cp[r * G:(r + 1) * G, :], wo_ref[:, :],
                            (((1,), (0,)), ((), ())),
                            preferred_element_type=jnp.float32)
        for j in range(J):
            b = R * j + r
            out_ref[0, b * BLK:(b + 1) * BLK, :] = p[j * BLK:(j + 1) * BLK, :]

    bsem = pltpu.get_barrier_semaphore()
    for nbr in (left, right):
        pl.semaphore_signal(bsem, inc=1, device_id=(nbr,),
                            device_id_type=pl.DeviceIdType.MESH)
    pl.semaphore_wait(bsem, 2)

    sb[0] = out_ref[0, pl.ds(my * CH, CH), :].astype(jnp.bfloat16)
    for st in range(N_DEV - 1):
        rdma = pltpu.make_async_remote_copy(
            src_ref=sb.at[st],
            dst_ref=rs.at[st],
            send_sem=send_sems.at[st],
            recv_sem=recv_sems.at[st],
            device_id=(right,),
            device_id_type=pl.DeviceIdType.MESH,
        )
        rdma.start()
        rdma.wait()
        rc = jnp.mod(my - st - 1, N_DEV)
        acc = rs[st].astype(jnp.float32) + out_ref[0, pl.ds(rc * CH, CH), :]
        if st < N_DEV - 2:
            sb[st + 1] = acc.astype(jnp.bfloat16)
        else:
            out_ref[0, pl.ds(rc * CH, CH), :] = acc
            sb[N_DEV - 1] = acc.astype(jnp.bfloat16)

    for t in range(N_DEV - 1):
        rdma = pltpu.make_async_remote_copy(
            src_ref=sb.at[N_DEV - 1] if t == 0 else ag.at[t - 1],
            dst_ref=ag.at[t],
            send_sem=send_sems.at[N_DEV - 1 + t],
            recv_sem=recv_sems.at[N_DEV - 1 + t],
            device_id=(right,),
            device_id_type=pl.DeviceIdType.MESH,
        )
        rdma.start()
        rdma.wait()
        rc = jnp.mod(my - t, N_DEV)
        out_ref[0, pl.ds(rc * CH, CH), :] = ag[t].astype(jnp.float32)


def kernel(x, Wq, K_ext, V_ext, Wo):
    my = lax.axis_index("i")
    xb = x[0].astype(jnp.bfloat16)
    wq = Wq.astype(jnp.bfloat16)
    wo = Wo.astype(jnp.bfloat16)
    kh = lax.dynamic_slice_in_dim(K_ext[0], my * HN, HN, axis=1)
    vh = lax.dynamic_slice_in_dim(V_ext[0], my * HN, HN, axis=1)
    kb = kh.reshape(S, HN * DH).astype(jnp.bfloat16)
    vb = vh.reshape(S, HN * DH).astype(jnp.bfloat16)

    return pl.pallas_call(
        _body,
        out_shape=jax.ShapeDtypeStruct((1, S, D), jnp.float32),
        in_specs=[pl.BlockSpec(memory_space=pltpu.VMEM)] * 5,
        out_specs=pl.BlockSpec(memory_space=pltpu.VMEM),
        scratch_shapes=[
            pltpu.VMEM((S, D), jnp.bfloat16),
            pltpu.VMEM((S, D), jnp.bfloat16),
            pltpu.VMEM((S, D), jnp.bfloat16),
            pltpu.VMEM((S, D), jnp.bfloat16),
            pltpu.VMEM((N_DEV, CH, D), jnp.bfloat16),
            pltpu.VMEM((N_DEV - 1, CH, D), jnp.bfloat16),
            pltpu.VMEM((N_DEV - 1, CH, D), jnp.bfloat16),
            pltpu.SemaphoreType.DMA((2 * (N_DEV - 1),)),
            pltpu.SemaphoreType.DMA((2 * (N_DEV - 1),)),
        ],
        compiler_params=pltpu.CompilerParams(
            collective_id=0, vmem_limit_bytes=63 * 1024 * 1024),
    )(xb, wq, kb, vb, wo)


# device time: 114403 ns/iter; 1.8808x vs baseline; 1.2918x over previous
import jax
import jax.numpy as jnp
from jax import lax
from jax.experimental import pallas as pl
from jax.experimental.pallas import tpu as pltpu

N_DEV = 4
S = 2048
D = 1024
HN = 8
DH = 128
BLK = 64
NB = S // BLK
R = 4
J = NB // R
G = J * BLK
CH = S // N_DEV
SCALE = 0.08838834764831843


def _body(x_ref, wq_ref, k_ref, v_ref, wo_ref, out_ref,
          qp, kp, vp, cp, sba, sbb, rsa, rsb, aga, agb,
          send_sems, recv_sems, send_sems2, recv_sems2):
    my = lax.axis_index("i")
    right = jnp.mod(my + 1, N_DEV)
    left = jnp.mod(my + N_DEV - 1, N_DEV)

    for b in range(NB):
        r, j = b % R, b // R
        d0 = r * G + j * BLK
        s0 = b * BLK
        kp[d0:d0 + BLK, :] = k_ref[s0:s0 + BLK, :]
        vp[d0:d0 + BLK, :] = v_ref[s0:s0 + BLK, :]

    for c in range(N_DEV):
        qf = lax.dot_general(
            x_ref[c * CH:(c + 1) * CH, :], wq_ref[:, :],
            (((1,), (0,)), ((), ())), preferred_element_type=jnp.float32)
        qb = (qf * SCALE).astype(jnp.bfloat16)
        for m in range(CH // BLK):
            b = (CH // BLK) * c + m
            r, j = b % R, b // R
            d0 = r * G + j * BLK
            qp[d0:d0 + BLK, :] = qb[m * BLK:(m + 1) * BLK, :]

    for r in range(R):
        for h in range(HN):
            rows = slice(r * G, (r + 1) * G)
            cols = slice(h * DH, (h + 1) * DH)
            q = qp[rows, cols]
            k = kp[rows, cols]
            v = vp[rows, cols]
            s = lax.dot_general(q, k, (((1,), (1,)), ((), ())),
                                preferred_element_type=jnp.float32)
            mx = jnp.max(s, axis=1, keepdims=True)
            w = jnp.exp(s - mx)
            w = w / jnp.sum(w, axis=1, keepdims=True)
            ctx = lax.dot_general(w.astype(jnp.bfloat16), v,
                                  (((1,), (0,)), ((), ())),
                                  preferred_element_type=jnp.float32)
            cp[rows, cols] = ctx.astype(jnp.bfloat16)

    for r in range(R):
        p = lax.dot_general(cp[r * G:(r + 1) * G, :], wo_ref[:, :],
                            (((1,), (0,)), ((), ())),
                            preferred_element_type=jnp.float32)
        for j in range(J):
            b = R * j + r
            out_ref[0, b * BLK:(b + 1) * BLK, :] = p[j * BLK:(j + 1) * BLK, :]

    bsem = pltpu.get_barrier_semaphore()
    for nbr in (left, right):
        pl.semaphore_signal(bsem, inc=1, device_id=(nbr,),
                            device_id_type=pl.DeviceIdType.MESH)
    pl.semaphore_wait(bsem, 2)

    D2 = D // 2
    colA = slice(0, D2)
    colB = slice(D2, D)

    sba[0] = out_ref[0, pl.ds(my * CH, CH), colA].astype(jnp.bfloat16)
    sbb[0] = out_ref[0, pl.ds(my * CH, CH), colB].astype(jnp.bfloat16)
    for st in range(N_DEV - 1):
        ra = pltpu.make_async_remote_copy(
            src_ref=sba.at[st], dst_ref=rsa.at[st],
            send_sem=send_sems.at[st], recv_sem=recv_sems.at[st],
            device_id=(right,), device_id_type=pl.DeviceIdType.MESH)
        rb = pltpu.make_async_remote_copy(
            src_ref=sbb.at[st], dst_ref=rsb.at[st],
            send_sem=send_sems2.at[st], recv_sem=recv_sems2.at[st],
            device_id=(left,), device_id_type=pl.DeviceIdType.MESH)
        ra.start()
        rb.start()
        ra.wait()
        rb.wait()
        rca = jnp.mod(my - st - 1, N_DEV)
        rcb = jnp.mod(my + st + 1, N_DEV)
        acca = rsa[st].astype(jnp.float32) + out_ref[0, pl.ds(rca * CH, CH), colA]
        accb = rsb[st].astype(jnp.float32) + out_ref[0, pl.ds(rcb * CH, CH), colB]
        if st < N_DEV - 2:
            sba[st + 1] = acca.astype(jnp.bfloat16)
            sbb[st + 1] = accb.astype(jnp.bfloat16)
        else:
            out_ref[0, pl.ds(rca * CH, CH), colA] = acca
            out_ref[0, pl.ds(rcb * CH, CH), colB] = accb
            sba[N_DEV - 1] = acca.astype(jnp.bfloat16)
            sbb[N_DEV - 1] = accb.astype(jnp.bfloat16)

    for t in range(N_DEV - 1):
        ra = pltpu.make_async_remote_copy(
            src_ref=sba.at[N_DEV - 1] if t == 0 else aga.at[t - 1],
            dst_ref=aga.at[t],
            send_sem=send_sems.at[N_DEV - 1 + t],
            recv_sem=recv_sems.at[N_DEV - 1 + t],
            device_id=(right,), device_id_type=pl.DeviceIdType.MESH)
        rb = pltpu.make_async_remote_copy(
            src_ref=sbb.at[N_DEV - 1] if t == 0 else agb.at[t - 1],
            dst_ref=agb.at[t],
            send_sem=send_sems2.at[N_DEV - 1 + t],
            recv_sem=recv_sems2.at[N_DEV - 1 + t],
            device_id=(left,), device_id_type=pl.DeviceIdType.MESH)
        ra.start()
        rb.start()
        ra.wait()
        rb.wait()
        rca = jnp.mod(my - t, N_DEV)
        rcb = jnp.mod(my + t, N_DEV)
        out_ref[0, pl.ds(rca * CH, CH), colA] = aga[t].astype(jnp.float32)
        out_ref[0, pl.ds(rcb * CH, CH), colB] = agb[t].astype(jnp.float32)


def kernel(x, Wq, K_ext, V_ext, Wo):
    my = lax.axis_index("i")
    xb = x[0].astype(jnp.bfloat16)
    wq = Wq.astype(jnp.bfloat16)
    wo = Wo.astype(jnp.bfloat16)
    kh = lax.dynamic_slice_in_dim(K_ext[0], my * HN, HN, axis=1)
    vh = lax.dynamic_slice_in_dim(V_ext[0], my * HN, HN, axis=1)
    kb = kh.reshape(S, HN * DH).astype(jnp.bfloat16)
    vb = vh.reshape(S, HN * DH).astype(jnp.bfloat16)

    return pl.pallas_call(
        _body,
        out_shape=jax.ShapeDtypeStruct((1, S, D), jnp.float32),
        in_specs=[pl.BlockSpec(memory_space=pltpu.VMEM)] * 5,
        out_specs=pl.BlockSpec(memory_space=pltpu.VMEM),
        scratch_shapes=[
            pltpu.VMEM((S, D), jnp.bfloat16),
            pltpu.VMEM((S, D), jnp.bfloat16),
            pltpu.VMEM((S, D), jnp.bfloat16),
            pltpu.VMEM((S, D), jnp.bfloat16),
            pltpu.VMEM((N_DEV, CH, D // 2), jnp.bfloat16),
            pltpu.VMEM((N_DEV, CH, D // 2), jnp.bfloat16),
            pltpu.VMEM((N_DEV - 1, CH, D // 2), jnp.bfloat16),
            pltpu.VMEM((N_DEV - 1, CH, D // 2), jnp.bfloat16),
            pltpu.VMEM((N_DEV - 1, CH, D // 2), jnp.bfloat16),
            pltpu.VMEM((N_DEV - 1, CH, D // 2), jnp.bfloat16),
            pltpu.SemaphoreType.DMA((2 * (N_DEV - 1),)),
            pltpu.SemaphoreType.DMA((2 * (N_DEV - 1),)),
            pltpu.SemaphoreType.DMA((2 * (N_DEV - 1),)),
            pltpu.SemaphoreType.DMA((2 * (N_DEV - 1),)),
        ],
        compiler_params=pltpu.CompilerParams(
            collective_id=0, vmem_limit_bytes=63 * 1024 * 1024),
    )(xb, wq, kb, vb, wo)


# device time: 111554 ns/iter; 1.9288x vs baseline; 1.0255x over previous
import jax
import jax.numpy as jnp
from jax import lax
from jax.experimental import pallas as pl
from jax.experimental.pallas import tpu as pltpu

N_DEV = 4
S = 2048
D = 1024
HN = 8
DH = 128
BLK = 64
NB = S // BLK
R = 4
J = NB // R
G = J * BLK
CH = S // N_DEV
SCALE = 0.08838834764831843


def _body(x_ref, wq_ref, k_ref, v_ref, wo_ref, out_ref,
          qp, kp, vp, cp, cg, sba, sbb, rsa, rsb, aga, agb,
          send_sems, recv_sems, send_sems2, recv_sems2):
    my = lax.axis_index("i")
    right = jnp.mod(my + 1, N_DEV)
    left = jnp.mod(my + N_DEV - 1, N_DEV)

    for b in range(NB):
        r, j = b % R, b // R
        d0 = r * G + j * BLK
        s0 = b * BLK
        kp[d0:d0 + BLK, :] = k_ref[s0:s0 + BLK, :]
        vp[d0:d0 + BLK, :] = v_ref[s0:s0 + BLK, :]

    for c in range(N_DEV):
        qf = lax.dot_general(
            x_ref[c * CH:(c + 1) * CH, :], wq_ref[:, :],
            (((1,), (0,)), ((), ())), preferred_element_type=jnp.float32)
        qb = (qf * SCALE).astype(jnp.bfloat16)
        for m in range(CH // BLK):
            b = (CH // BLK) * c + m
            r, j = b % R, b // R
            d0 = r * G + j * BLK
            qp[d0:d0 + BLK, :] = qb[m * BLK:(m + 1) * BLK, :]

    for r in range(R):
        for h in range(HN):
            rows = slice(r * G, (r + 1) * G)
            cols = slice(h * DH, (h + 1) * DH)
            q = qp[rows, cols]
            k = kp[rows, cols]
            v = vp[rows, cols]
            s = lax.dot_general(q, k, (((1,), (1,)), ((), ())),
                                preferred_element_type=jnp.float32)
            mx = jnp.max(s, axis=1, keepdims=True)
            w = jnp.exp(s - mx)
            w = w / jnp.sum(w, axis=1, keepdims=True)
            ctx = lax.dot_general(w.astype(jnp.bfloat16), v,
                                  (((1,), (0,)), ((), ())),
                                  preferred_element_type=jnp.float32)
            cp[rows, cols] = ctx.astype(jnp.bfloat16)

    D2 = D // 2
    colA = slice(0, D2)
    colB = slice(D2, D)

    def outproj(g):
        for r in range(R):
            cg[r * 128:(r + 1) * 128, :] = cp[pl.ds(r * G + g * 128, 128), :]
        p = lax.dot_general(cg[:, :], wo_ref[:, :],
                            (((1,), (0,)), ((), ())),
                            preferred_element_type=jnp.float32)
        for m in range(CH // BLK):
            srow = (m % R) * 128 + (m // R) * BLK
            out_ref[0, pl.ds(g * CH + m * BLK, BLK), :] = \
                p[srow:srow + BLK, :]

    def rs_hop(st):
        ra = pltpu.make_async_remote_copy(
            src_ref=sba.at[st], dst_ref=rsa.at[st],
            send_sem=send_sems.at[st], recv_sem=recv_sems.at[st],
            device_id=(right,), device_id_type=pl.DeviceIdType.MESH)
        rb = pltpu.make_async_remote_copy(
            src_ref=sbb.at[st], dst_ref=rsb.at[st],
            send_sem=send_sems2.at[st], recv_sem=recv_sems2.at[st],
            device_id=(left,), device_id_type=pl.DeviceIdType.MESH)
        ra.start()
        rb.start()
        return ra, rb

    def rs_finish(st, ra, rb):
        ra.wait()
        rb.wait()
        rca = jnp.mod(my - st - 1, N_DEV)
        rcb = jnp.mod(my + st + 1, N_DEV)
        acca = rsa[st].astype(jnp.float32) + out_ref[0, pl.ds(rca * CH, CH), colA]
        accb = rsb[st].astype(jnp.float32) + out_ref[0, pl.ds(rcb * CH, CH), colB]
        if st < N_DEV - 2:
            sba[st + 1] = acca.astype(jnp.bfloat16)
            sbb[st + 1] = accb.astype(jnp.bfloat16)
        else:
            out_ref[0, pl.ds(rca * CH, CH), colA] = acca
            out_ref[0, pl.ds(rcb * CH, CH), colB] = accb
            sba[N_DEV - 1] = acca.astype(jnp.bfloat16)
            sbb[N_DEV - 1] = accb.astype(jnp.bfloat16)

    outproj(my)
    sba[0] = out_ref[0, pl.ds(my * CH, CH), colA].astype(jnp.bfloat16)
    sbb[0] = out_ref[0, pl.ds(my * CH, CH), colB].astype(jnp.bfloat16)

    bsem = pltpu.get_barrier_semaphore()
    for nbr in (left, right):
        pl.semaphore_signal(bsem, inc=1, device_id=(nbr,),
                            device_id_type=pl.DeviceIdType.MESH)
    pl.semaphore_wait(bsem, 2)

    h0 = rs_hop(0)
    outproj(jnp.mod(my + 1, N_DEV))
    outproj(jnp.mod(my + 3, N_DEV))
    rs_finish(0, *h0)
    h1 = rs_hop(1)
    outproj(jnp.mod(my + 2, N_DEV))
    rs_finish(1, *h1)
    h2 = rs_hop(2)
    rs_finish(2, *h2)

    for t in range(N_DEV - 1):
        ra = pltpu.make_async_remote_copy(
            src_ref=sba.at[N_DEV - 1] if t == 0 else aga.at[t - 1],
            dst_ref=aga.at[t],
            send_sem=send_sems.at[N_DEV - 1 + t],
            recv_sem=recv_sems.at[N_DEV - 1 + t],
            device_id=(right,), device_id_type=pl.DeviceIdType.MESH)
        rb = pltpu.make_async_remote_copy(
            src_ref=sbb.at[N_DEV - 1] if t == 0 else agb.at[t - 1],
            dst_ref=agb.at[t],
            send_sem=send_sems2.at[N_DEV - 1 + t],
            recv_sem=recv_sems2.at[N_DEV - 1 + t],
            device_id=(left,), device_id_type=pl.DeviceIdType.MESH)
        ra.start()
        rb.start()
        ra.wait()
        rb.wait()
        rca = jnp.mod(my - t, N_DEV)
        rcb = jnp.mod(my + t, N_DEV)
        out_ref[0, pl.ds(rca * CH, CH), colA] = aga[t].astype(jnp.float32)
        out_ref[0, pl.ds(rcb * CH, CH), colB] = agb[t].astype(jnp.float32)


def kernel(x, Wq, K_ext, V_ext, Wo):
    my = lax.axis_index("i")
    xb = x[0].astype(jnp.bfloat16)
    wq = Wq.astype(jnp.bfloat16)
    wo = Wo.astype(jnp.bfloat16)
    kh = lax.dynamic_slice_in_dim(K_ext[0], my * HN, HN, axis=1)
    vh = lax.dynamic_slice_in_dim(V_ext[0], my * HN, HN, axis=1)
    kb = kh.reshape(S, HN * DH).astype(jnp.bfloat16)
    vb = vh.reshape(S, HN * DH).astype(jnp.bfloat16)

    return pl.pallas_call(
        _body,
        out_shape=jax.ShapeDtypeStruct((1, S, D), jnp.float32),
        in_specs=[pl.BlockSpec(memory_space=pltpu.VMEM)] * 5,
        out_specs=pl.BlockSpec(memory_space=pltpu.VMEM),
        scratch_shapes=[
            pltpu.VMEM((S, D), jnp.bfloat16),
            pltpu.VMEM((S, D), jnp.bfloat16),
            pltpu.VMEM((S, D), jnp.bfloat16),
            pltpu.VMEM((S, D), jnp.bfloat16),
            pltpu.VMEM((CH, D), jnp.bfloat16),
            pltpu.VMEM((N_DEV, CH, D // 2), jnp.bfloat16),
            pltpu.VMEM((N_DEV, CH, D // 2), jnp.bfloat16),
            pltpu.VMEM((N_DEV - 1, CH, D // 2), jnp.bfloat16),
            pltpu.VMEM((N_DEV - 1, CH, D // 2), jnp.bfloat16),
            pltpu.VMEM((N_DEV - 1, CH, D // 2), jnp.bfloat16),
            pltpu.VMEM((N_DEV - 1, CH, D // 2), jnp.bfloat16),
            pltpu.SemaphoreType.DMA((2 * (N_DEV - 1),)),
            pltpu.SemaphoreType.DMA((2 * (N_DEV - 1),)),
            pltpu.SemaphoreType.DMA((2 * (N_DEV - 1),)),
            pltpu.SemaphoreType.DMA((2 * (N_DEV - 1),)),
        ],
        compiler_params=pltpu.CompilerParams(
            collective_id=0, vmem_limit_bytes=63 * 1024 * 1024),
    )(xb, wq, kb, vb, wo)


# device time: 100582 ns/iter; 2.1392x vs baseline; 1.1091x over previous
import jax
import jax.numpy as jnp
from jax import lax
from jax.experimental import pallas as pl
from jax.experimental.pallas import tpu as pltpu

N_DEV = 4
S = 2048
D = 1024
HN = 8
DH = 128
BLK = 64
NB = S // BLK
R = 4
J = NB // R
G = J * BLK
CH = S // N_DEV
SCALE = 0.08838834764831843


def _body(x_ref, wq_ref, k_ref, v_ref, wo_ref, out_ref,
          qp, kp, vp, cp, cg, wqb, wob, sba, sbb, rsa, rsb, aga, agb,
          send_sems, recv_sems, send_sems2, recv_sems2):
    my = lax.axis_index("i")
    right = jnp.mod(my + 1, N_DEV)
    left = jnp.mod(my + N_DEV - 1, N_DEV)

    wqb[:, :] = wq_ref[:, :].astype(jnp.bfloat16)
    wob[:, :] = wo_ref[:, :].astype(jnp.bfloat16)

    for b in range(NB):
        r, j = b % R, b // R
        d0 = r * G + j * BLK
        s0 = b * BLK
        kp[d0:d0 + BLK, :] = k_ref[s0:s0 + BLK, :]
        vp[d0:d0 + BLK, :] = v_ref[s0:s0 + BLK, :]

    for c in range(N_DEV):
        qf = lax.dot_general(
            x_ref[c * CH:(c + 1) * CH, :].astype(jnp.bfloat16), wqb[:, :],
            (((1,), (0,)), ((), ())), preferred_element_type=jnp.float32)
        qb = (qf * SCALE).astype(jnp.bfloat16)
        for m in range(CH // BLK):
            b = (CH // BLK) * c + m
            r, j = b % R, b // R
            d0 = r * G + j * BLK
            qp[d0:d0 + BLK, :] = qb[m * BLK:(m + 1) * BLK, :]

    for r in range(R):
        for h in range(HN):
            rows = slice(r * G, (r + 1) * G)
            cols = slice(h * DH, (h + 1) * DH)
            q = qp[rows, cols]
            k = kp[rows, cols]
            v = vp[rows, cols]
            s = lax.dot_general(q, k, (((1,), (1,)), ((), ())),
                                preferred_element_type=jnp.float32)
            w = jnp.exp(s)
            sums = jnp.sum(w, axis=1, keepdims=True)
            ctx = lax.dot_general(w.astype(jnp.bfloat16), v,
                                  (((1,), (0,)), ((), ())),
                                  preferred_element_type=jnp.float32)
            cp[rows, cols] = (ctx * (1.0 / sums)).astype(jnp.bfloat16)

    D2 = D // 2
    colA = slice(0, D2)
    colB = slice(D2, D)

    def outproj(g):
        for r in range(R):
            cg[r * 128:(r + 1) * 128, :] = cp[pl.ds(r * G + g * 128, 128), :]
        p = lax.dot_general(cg[:, :], wob[:, :],
                            (((1,), (0,)), ((), ())),
                            preferred_element_type=jnp.float32)
        for m in range(CH // BLK):
            srow = (m % R) * 128 + (m // R) * BLK
            out_ref[0, pl.ds(g * CH + m * BLK, BLK), :] = \
                p[srow:srow + BLK, :]

    def rs_hop(st):
        ra = pltpu.make_async_remote_copy(
            src_ref=sba.at[st], dst_ref=rsa.at[st],
            send_sem=send_sems.at[st], recv_sem=recv_sems.at[st],
            device_id=(right,), device_id_type=pl.DeviceIdType.MESH)
        rb = pltpu.make_async_remote_copy(
            src_ref=sbb.at[st], dst_ref=rsb.at[st],
            send_sem=send_sems2.at[st], recv_sem=recv_sems2.at[st],
            device_id=(left,), device_id_type=pl.DeviceIdType.MESH)
        ra.start()
        rb.start()
        return ra, rb

    def rs_finish(st, ra, rb):
        ra.wait()
        rb.wait()
        rca = jnp.mod(my - st - 1, N_DEV)
        rcb = jnp.mod(my + st + 1, N_DEV)
        acca = rsa[st].astype(jnp.float32) + out_ref[0, pl.ds(rca * CH, CH), colA]
        accb = rsb[st].astype(jnp.float32) + out_ref[0, pl.ds(rcb * CH, CH), colB]
        if st < N_DEV - 2:
            sba[st + 1] = acca.astype(jnp.bfloat16)
            sbb[st + 1] = accb.astype(jnp.bfloat16)
        else:
            out_ref[0, pl.ds(rca * CH, CH), colA] = acca
            out_ref[0, pl.ds(rcb * CH, CH), colB] = accb
            sba[N_DEV - 1] = acca.astype(jnp.bfloat16)
            sbb[N_DEV - 1] = accb.astype(jnp.bfloat16)

    outproj(my)
    sba[0] = out_ref[0, pl.ds(my * CH, CH), colA].astype(jnp.bfloat16)
    sbb[0] = out_ref[0, pl.ds(my * CH, CH), colB].astype(jnp.bfloat16)

    bsem = pltpu.get_barrier_semaphore()
    for nbr in (left, right):
        pl.semaphore_signal(bsem, inc=1, device_id=(nbr,),
                            device_id_type=pl.DeviceIdType.MESH)
    pl.semaphore_wait(bsem, 2)

    h0 = rs_hop(0)
    outproj(jnp.mod(my + 1, N_DEV))
    outproj(jnp.mod(my + 3, N_DEV))
    rs_finish(0, *h0)
    h1 = rs_hop(1)
    outproj(jnp.mod(my + 2, N_DEV))
    rs_finish(1, *h1)
    h2 = rs_hop(2)
    rs_finish(2, *h2)

    for t in range(N_DEV - 1):
        ra = pltpu.make_async_remote_copy(
            src_ref=sba.at[N_DEV - 1] if t == 0 else aga.at[t - 1],
            dst_ref=aga.at[t],
            send_sem=send_sems.at[N_DEV - 1 + t],
            recv_sem=recv_sems.at[N_DEV - 1 + t],
            device_id=(right,), device_id_type=pl.DeviceIdType.MESH)
        rb = pltpu.make_async_remote_copy(
            src_ref=sbb.at[N_DEV - 1] if t == 0 else agb.at[t - 1],
            dst_ref=agb.at[t],
            send_sem=send_sems2.at[N_DEV - 1 + t],
            recv_sem=recv_sems2.at[N_DEV - 1 + t],
            device_id=(left,), device_id_type=pl.DeviceIdType.MESH)
        ra.start()
        rb.start()
        ra.wait()
        rb.wait()
        rca = jnp.mod(my - t, N_DEV)
        rcb = jnp.mod(my + t, N_DEV)
        out_ref[0, pl.ds(rca * CH, CH), colA] = aga[t].astype(jnp.float32)
        out_ref[0, pl.ds(rcb * CH, CH), colB] = agb[t].astype(jnp.float32)


def kernel(x, Wq, K_ext, V_ext, Wo):
    my = lax.axis_index("i")
    xb = x[0]
    kh = lax.dynamic_slice_in_dim(K_ext[0], my * HN, HN, axis=1)
    vh = lax.dynamic_slice_in_dim(V_ext[0], my * HN, HN, axis=1)
    kb = kh.reshape(S, HN * DH).astype(jnp.bfloat16)
    vb = vh.reshape(S, HN * DH).astype(jnp.bfloat16)

    return pl.pallas_call(
        _body,
        out_shape=jax.ShapeDtypeStruct((1, S, D), jnp.float32),
        in_specs=[pl.BlockSpec(memory_space=pltpu.VMEM)] * 5,
        out_specs=pl.BlockSpec(memory_space=pltpu.VMEM),
        scratch_shapes=[
            pltpu.VMEM((S, D), jnp.bfloat16),
            pltpu.VMEM((S, D), jnp.bfloat16),
            pltpu.VMEM((S, D), jnp.bfloat16),
            pltpu.VMEM((S, D), jnp.bfloat16),
            pltpu.VMEM((CH, D), jnp.bfloat16),
            pltpu.VMEM((D, D), jnp.bfloat16),
            pltpu.VMEM((D, D), jnp.bfloat16),
            pltpu.VMEM((N_DEV, CH, D // 2), jnp.bfloat16),
            pltpu.VMEM((N_DEV, CH, D // 2), jnp.bfloat16),
            pltpu.VMEM((N_DEV - 1, CH, D // 2), jnp.bfloat16),
            pltpu.VMEM((N_DEV - 1, CH, D // 2), jnp.bfloat16),
            pltpu.VMEM((N_DEV - 1, CH, D // 2), jnp.bfloat16),
            pltpu.VMEM((N_DEV - 1, CH, D // 2), jnp.bfloat16),
            pltpu.SemaphoreType.DMA((2 * (N_DEV - 1),)),
            pltpu.SemaphoreType.DMA((2 * (N_DEV - 1),)),
            pltpu.SemaphoreType.DMA((2 * (N_DEV - 1),)),
            pltpu.SemaphoreType.DMA((2 * (N_DEV - 1),)),
        ],
        compiler_params=pltpu.CompilerParams(
            collective_id=0, vmem_limit_bytes=63 * 1024 * 1024),
    )(xb, Wq, kb, vb, Wo)


# device time: 100389 ns/iter; 2.1433x vs baseline; 1.0019x over previous
import jax
import jax.numpy as jnp
from jax import lax
from jax.experimental import pallas as pl
from jax.experimental.pallas import tpu as pltpu

N_DEV = 4
S = 2048
D = 1024
HN = 8
DH = 128
BLK = 64
NB = S // BLK
R = 4
J = NB // R
G = J * BLK
CH = S // N_DEV
SCALE = 0.08838834764831843


def _body(x_ref, wq_ref, k_ref, v_ref, wo_ref, out_ref,
          qp, kp, vp, cp, cg, wqb, wob, sba, sbb, rsa, rsb, aga, agb,
          send_sems, recv_sems, send_sems2, recv_sems2):
    my = lax.axis_index("i")
    right = jnp.mod(my + 1, N_DEV)
    left = jnp.mod(my + N_DEV - 1, N_DEV)

    wqb[:, :] = wq_ref[:, :].astype(jnp.bfloat16)
    wob[:, :] = wo_ref[:, :].astype(jnp.bfloat16)

    for b in range(NB):
        r, j = b % R, b // R
        d0 = r * G + j * BLK
        s0 = b * BLK
        kp[d0:d0 + BLK, :] = k_ref[s0:s0 + BLK, :]
        vp[d0:d0 + BLK, :] = v_ref[s0:s0 + BLK, :]

    for c in range(N_DEV):
        qf = lax.dot_general(
            x_ref[c * CH:(c + 1) * CH, :].astype(jnp.bfloat16), wqb[:, :],
            (((1,), (0,)), ((), ())), preferred_element_type=jnp.float32)
        qb = (qf * SCALE).astype(jnp.bfloat16)
        for m in range(CH // BLK):
            b = (CH // BLK) * c + m
            r, j = b % R, b // R
            d0 = r * G + j * BLK
            qp[d0:d0 + BLK, :] = qb[m * BLK:(m + 1) * BLK, :]

    for r in range(R):
        for h in range(HN):
            rows = slice(r * G, (r + 1) * G)
            cols = slice(h * DH, (h + 1) * DH)
            q = qp[rows, cols]
            k = kp[rows, cols]
            v = vp[rows, cols]
            s = lax.dot_general(q, k, (((1,), (1,)), ((), ())),
                                preferred_element_type=jnp.float32)
            w = jnp.exp(s).astype(jnp.bfloat16)
            sums = jnp.sum(w.astype(jnp.float32), axis=1, keepdims=True)
            ctx = lax.dot_general(w, v, (((1,), (0,)), ((), ())),
                                  preferred_element_type=jnp.float32)
            cp[rows, cols] = (ctx * (1.0 / sums)).astype(jnp.bfloat16)

    D2 = D // 2
    colA = slice(0, D2)
    colB = slice(D2, D)

    def outproj(g):
        for r in range(R):
            cg[r * 128:(r + 1) * 128, :] = cp[pl.ds(r * G + g * 128, 128), :]
        p = lax.dot_general(cg[:, :], wob[:, :],
                            (((1,), (0,)), ((), ())),
                            preferred_element_type=jnp.float32)
        for m in range(CH // BLK):
            srow = (m % R) * 128 + (m // R) * BLK
            out_ref[0, pl.ds(g * CH + m * BLK, BLK), :] = \
                p[srow:srow + BLK, :]

    def rs_hop(st):
        ra = pltpu.make_async_remote_copy(
            src_ref=sba.at[st], dst_ref=rsa.at[st],
            send_sem=send_sems.at[st], recv_sem=recv_sems.at[st],
            device_id=(right,), device_id_type=pl.DeviceIdType.MESH)
        rb = pltpu.make_async_remote_copy(
            src_ref=sbb.at[st], dst_ref=rsb.at[st],
            send_sem=send_sems2.at[st], recv_sem=recv_sems2.at[st],
            device_id=(left,), device_id_type=pl.DeviceIdType.MESH)
        ra.start()
        rb.start()
        return ra, rb

    def rs_finish(st, ra, rb):
        ra.wait()
        rb.wait()
        rca = jnp.mod(my - st - 1, N_DEV)
        rcb = jnp.mod(my + st + 1, N_DEV)
        acca = rsa[st].astype(jnp.float32) + out_ref[0, pl.ds(rca * CH, CH), colA]
        accb = rsb[st].astype(jnp.float32) + out_ref[0, pl.ds(rcb * CH, CH), colB]
        if st < N_DEV - 2:
            sba[st + 1] = acca.astype(jnp.bfloat16)
            sbb[st + 1] = accb.astype(jnp.bfloat16)
        else:
            out_ref[0, pl.ds(rca * CH, CH), colA] = acca
            out_ref[0, pl.ds(rcb * CH, CH), colB] = accb
            sba[N_DEV - 1] = acca.astype(jnp.bfloat16)
            sbb[N_DEV - 1] = accb.astype(jnp.bfloat16)

    outproj(my)
    sba[0] = out_ref[0, pl.ds(my * CH, CH), colA].astype(jnp.bfloat16)
    sbb[0] = out_ref[0, pl.ds(my * CH, CH), colB].astype(jnp.bfloat16)

    bsem = pltpu.get_barrier_semaphore()
    for nbr in (left, right):
        pl.semaphore_signal(bsem, inc=1, device_id=(nbr,),
                            device_id_type=pl.DeviceIdType.MESH)
    pl.semaphore_wait(bsem, 2)

    h0 = rs_hop(0)
    outproj(jnp.mod(my + 1, N_DEV))
    outproj(jnp.mod(my + 3, N_DEV))
    rs_finish(0, *h0)
    h1 = rs_hop(1)
    outproj(jnp.mod(my + 2, N_DEV))
    rs_finish(1, *h1)
    h2 = rs_hop(2)
    rs_finish(2, *h2)

    for t in range(N_DEV - 1):
        ra = pltpu.make_async_remote_copy(
            src_ref=sba.at[N_DEV - 1] if t == 0 else aga.at[t - 1],
            dst_ref=aga.at[t],
            send_sem=send_sems.at[N_DEV - 1 + t],
            recv_sem=recv_sems.at[N_DEV - 1 + t],
            device_id=(right,), device_id_type=pl.DeviceIdType.MESH)
        rb = pltpu.make_async_remote_copy(
            src_ref=sbb.at[N_DEV - 1] if t == 0 else agb.at[t - 1],
            dst_ref=agb.at[t],
            send_sem=send_sems2.at[N_DEV - 1 + t],
            recv_sem=recv_sems2.at[N_DEV - 1 + t],
            device_id=(left,), device_id_type=pl.DeviceIdType.MESH)
        ra.start()
        rb.start()
        ra.wait()
        rb.wait()
        rca = jnp.mod(my - t, N_DEV)
        rcb = jnp.mod(my + t, N_DEV)
        out_ref[0, pl.ds(rca * CH, CH), colA] = aga[t].astype(jnp.float32)
        out_ref[0, pl.ds(rcb * CH, CH), colB] = agb[t].astype(jnp.float32)


def kernel(x, Wq, K_ext, V_ext, Wo):
    my = lax.axis_index("i")
    xb = x[0]
    kh = lax.dynamic_slice_in_dim(K_ext[0], my * HN, HN, axis=1)
    vh = lax.dynamic_slice_in_dim(V_ext[0], my * HN, HN, axis=1)
    kb = kh.reshape(S, HN * DH).astype(jnp.bfloat16)
    vb = vh.reshape(S, HN * DH).astype(jnp.bfloat16)

    return pl.pallas_call(
        _body,
        out_shape=jax.ShapeDtypeStruct((1, S, D), jnp.float32),
        in_specs=[pl.BlockSpec(memory_space=pltpu.VMEM)] * 5,
        out_specs=pl.BlockSpec(memory_space=pltpu.VMEM),
        scratch_shapes=[
            pltpu.VMEM((S, D), jnp.bfloat16),
            pltpu.VMEM((S, D), jnp.bfloat16),
            pltpu.VMEM((S, D), jnp.bfloat16),
            pltpu.VMEM((S, D), jnp.bfloat16),
            pltpu.VMEM((CH, D), jnp.bfloat16),
            pltpu.VMEM((D, D), jnp.bfloat16),
            pltpu.VMEM((D, D), jnp.bfloat16),
            pltpu.VMEM((N_DEV, CH, D // 2), jnp.bfloat16),
            pltpu.VMEM((N_DEV, CH, D // 2), jnp.bfloat16),
            pltpu.VMEM((N_DEV - 1, CH, D // 2), jnp.bfloat16),
            pltpu.VMEM((N_DEV - 1, CH, D // 2), jnp.bfloat16),
            pltpu.VMEM((N_DEV - 1, CH, D // 2), jnp.bfloat16),
            pltpu.VMEM((N_DEV - 1, CH, D // 2), jnp.bfloat16),
            pltpu.SemaphoreType.DMA((2 * (N_DEV - 1),)),
            pltpu.SemaphoreType.DMA((2 * (N_DEV - 1),)),
            pltpu.SemaphoreType.DMA((2 * (N_DEV - 1),)),
            pltpu.SemaphoreType.DMA((2 * (N_DEV - 1),)),
        ],
        compiler_params=pltpu.CompilerParams(
            collective_id=0, vmem_limit_bytes=63 * 1024 * 1024),
    )(xb, Wq, kb, vb, Wo)


# device time: 49202 ns/iter; 4.3731x vs baseline; 2.0403x over previous
import jax
import jax.numpy as jnp
from jax import lax
from jax.experimental import pallas as pl
from jax.experimental.pallas import tpu as pltpu

N_DEV = 4
S = 2048
D = 1024
HN = 8
DH = 128
BLK = 64
NB = S // BLK
R = 4
J = NB // R
G = J * BLK
CH = S // N_DEV
SCALE = 0.08838834764831843
_RING = True


def _body(x_ref, wq_ref, k_ref, v_ref, wo_ref, out_ref,
          qp, kp, vp, cp, cg, wqb, wob, sba, sbb, rsa, rsb, aga, agb,
          send_sems, recv_sems, send_sems2, recv_sems2):
    my = lax.axis_index("i")
    right = jnp.mod(my + 1, N_DEV)
    left = jnp.mod(my + N_DEV - 1, N_DEV)

    wqb[:, :] = wq_ref[:, :].astype(jnp.bfloat16)
    wob[:, :] = wo_ref[:, :].astype(jnp.bfloat16)

    for b in range(NB):
        r, j = b % R, b // R
        d0 = r * G + j * BLK
        s0 = b * BLK
        kp[d0:d0 + BLK, :] = k_ref[s0:s0 + BLK, :]
        vp[d0:d0 + BLK, :] = v_ref[s0:s0 + BLK, :]

    for c in range(N_DEV):
        qf = lax.dot_general(
            x_ref[c * CH:(c + 1) * CH, :].astype(jnp.bfloat16), wqb[:, :],
            (((1,), (0,)), ((), ())), preferred_element_type=jnp.float32)
        qb = (qf * SCALE).astype(jnp.bfloat16)
        for m in range(CH // BLK):
            b = (CH // BLK) * c + m
            r, j = b % R, b // R
            d0 = r * G + j * BLK
            qp[d0:d0 + BLK, :] = qb[m * BLK:(m + 1) * BLK, :]

    for r in range(R):
        for h in range(HN):
            rows = slice(r * G, (r + 1) * G)
            cols = slice(h * DH, (h + 1) * DH)
            q = qp[rows, cols]
            k = kp[rows, cols]
            v = vp[rows, cols]
            s = lax.dot_general(q, k, (((1,), (1,)), ((), ())),
                                preferred_element_type=jnp.float32)
            w = jnp.exp(s).astype(jnp.bfloat16)
            sums = jnp.sum(w.astype(jnp.float32), axis=1, keepdims=True)
            ctx = lax.dot_general(w, v, (((1,), (0,)), ((), ())),
                                  preferred_element_type=jnp.float32)
            cp[rows, cols] = (ctx * (1.0 / sums)).astype(jnp.bfloat16)

    D2 = D // 2
    colA = slice(0, D2)
    colB = slice(D2, D)

    def outproj(g):
        for r in range(R):
            cg[r * 128:(r + 1) * 128, :] = cp[pl.ds(r * G + g * 128, 128), :]
        p = lax.dot_general(cg[:, :], wob[:, :],
                            (((1,), (0,)), ((), ())),
                            preferred_element_type=jnp.float32)
        for m in range(CH // BLK):
            srow = (m % R) * 128 + (m // R) * BLK
            out_ref[0, pl.ds(g * CH + m * BLK, BLK), :] = \
                p[srow:srow + BLK, :]

    def rs_hop(st):
        ra = pltpu.make_async_remote_copy(
            src_ref=sba.at[st], dst_ref=rsa.at[st],
            send_sem=send_sems.at[st], recv_sem=recv_sems.at[st],
            device_id=(right,), device_id_type=pl.DeviceIdType.MESH)
        rb = pltpu.make_async_remote_copy(
            src_ref=sbb.at[st], dst_ref=rsb.at[st],
            send_sem=send_sems2.at[st], recv_sem=recv_sems2.at[st],
            device_id=(left,), device_id_type=pl.DeviceIdType.MESH)
        ra.start()
        rb.start()
        return ra, rb

    def rs_finish(st, ra, rb):
        ra.wait()
        rb.wait()
        rca = jnp.mod(my - st - 1, N_DEV)
        rcb = jnp.mod(my + st + 1, N_DEV)
        acca = rsa[st].astype(jnp.float32) + out_ref[0, pl.ds(rca * CH, CH), colA]
        accb = rsb[st].astype(jnp.float32) + out_ref[0, pl.ds(rcb * CH, CH), colB]
        if st < N_DEV - 2:
            sba[st + 1] = acca.astype(jnp.bfloat16)
            sbb[st + 1] = accb.astype(jnp.bfloat16)
        else:
            out_ref[0, pl.ds(rca * CH, CH), colA] = acca
            out_ref[0, pl.ds(rcb * CH, CH), colB] = accb
            sba[N_DEV - 1] = acca.astype(jnp.bfloat16)
            sbb[N_DEV - 1] = accb.astype(jnp.bfloat16)

    if not _RING:
        for gg in range(N_DEV):
            outproj(jnp.mod(my + gg, N_DEV))
        return

    outproj(my)
    sba[0] = out_ref[0, pl.ds(my * CH, CH), colA].astype(jnp.bfloat16)
    sbb[0] = out_ref[0, pl.ds(my * CH, CH), colB].astype(jnp.bfloat16)

    bsem = pltpu.get_barrier_semaphore()
    for nbr in (left, right):
        pl.semaphore_signal(bsem, inc=1, device_id=(nbr,),
                            device_id_type=pl.DeviceIdType.MESH)
    pl.semaphore_wait(bsem, 2)

    h0 = rs_hop(0)
    outproj(jnp.mod(my + 1, N_DEV))
    outproj(jnp.mod(my + 3, N_DEV))
    rs_finish(0, *h0)
    h1 = rs_hop(1)
    outproj(jnp.mod(my + 2, N_DEV))
    rs_finish(1, *h1)
    h2 = rs_hop(2)
    rs_finish(2, *h2)

    for t in range(N_DEV - 1):
        ra = pltpu.make_async_remote_copy(
            src_ref=sba.at[N_DEV - 1] if t == 0 else aga.at[t - 1],
            dst_ref=aga.at[t],
            send_sem=send_sems.at[N_DEV - 1 + t],
            recv_sem=recv_sems.at[N_DEV - 1 + t],
            device_id=(right,), device_id_type=pl.DeviceIdType.MESH)
        rb = pltpu.make_async_remote_copy(
            src_ref=sbb.at[N_DEV - 1] if t == 0 else agb.at[t - 1],
            dst_ref=agb.at[t],
            send_sem=send_sems2.at[N_DEV - 1 + t],
            recv_sem=recv_sems2.at[N_DEV - 1 + t],
            device_id=(left,), device_id_type=pl.DeviceIdType.MESH)
        ra.start()
        rb.start()
        ra.wait()
        rb.wait()
        rca = jnp.mod(my - t, N_DEV)
        rcb = jnp.mod(my + t, N_DEV)
        out_ref[0, pl.ds(rca * CH, CH), colA] = aga[t].astype(jnp.float32)
        out_ref[0, pl.ds(rcb * CH, CH), colB] = agb[t].astype(jnp.float32)


def kernel(x, Wq, K_ext, V_ext, Wo):
    my = lax.axis_index("i")
    xb = x[0]
    kh = lax.dynamic_slice_in_dim(K_ext[0], my * HN, HN, axis=1)
    vh = lax.dynamic_slice_in_dim(V_ext[0], my * HN, HN, axis=1)
    kb = kh.reshape(S, HN * DH).astype(jnp.bfloat16)
    vb = vh.reshape(S, HN * DH).astype(jnp.bfloat16)

    return pl.pallas_call(
        _body,
        out_shape=jax.ShapeDtypeStruct((1, S, D), jnp.float32),
        in_specs=[pl.BlockSpec(memory_space=pltpu.VMEM)] * 5,
        out_specs=pl.BlockSpec(memory_space=pltpu.VMEM),
        scratch_shapes=[
            pltpu.VMEM((S, D), jnp.bfloat16),
            pltpu.VMEM((S, D), jnp.bfloat16),
            pltpu.VMEM((S, D), jnp.bfloat16),
            pltpu.VMEM((S, D), jnp.bfloat16),
            pltpu.VMEM((CH, D), jnp.bfloat16),
            pltpu.VMEM((D, D), jnp.bfloat16),
            pltpu.VMEM((D, D), jnp.bfloat16),
            pltpu.VMEM((N_DEV, CH, D // 2), jnp.bfloat16),
            pltpu.VMEM((N_DEV, CH, D // 2), jnp.bfloat16),
            pltpu.VMEM((N_DEV - 1, CH, D // 2), jnp.bfloat16),
            pltpu.VMEM((N_DEV - 1, CH, D // 2), jnp.bfloat16),
            pltpu.VMEM((N_DEV - 1, CH, D // 2), jnp.bfloat16),
            pltpu.VMEM((N_DEV - 1, CH, D // 2), jnp.bfloat16),
            pltpu.SemaphoreType.DMA((2 * (N_DEV - 1),)),
            pltpu.SemaphoreType.DMA((2 * (N_DEV - 1),)),
            pltpu.SemaphoreType.DMA((2 * (N_DEV - 1),)),
            pltpu.SemaphoreType.DMA((2 * (N_DEV - 1),)),
        ],
        compiler_params=pltpu.CompilerParams(
            collective_id=0 if _RING else None,
            vmem_limit_bytes=63 * 1024 * 1024),
    )(xb, Wq, kb, vb, Wo)
